# Initial kernel scaffold; baseline (speedup 1.0000x reference)
import jax
import jax.numpy as jnp
from jax.experimental import pallas as pl


def kernel(x, embeddings):
    raise NotImplementedError("write your pallas kernel here")


# speedup vs baseline: 1.4600x; 1.4600x over previous
import jax
import jax.numpy as jnp
from jax import lax
from jax.experimental import pallas as pl
from jax.experimental.pallas import tpu as pltpu
from jax.experimental.pallas import tpu_sc as plsc

D = 256
K = 8192
N = 65536
TN = 1024
CC = 2048
EB = 1024
EPS = 1e-12
COMMIT = 0.25

NC = 2
NS = 16
NW = NC * NS
CK = 128
ROWS_PER_W = N // NW
CH_PER_W = ROWS_PER_W // CK


def _norm_body(e_ref, en_ref, enb_ref, c_ref):
    e = e_ref[...]
    nrm = jnp.sqrt(jnp.sum(e * e, axis=1, keepdims=True))
    en = e / jnp.maximum(nrm, EPS)
    en_ref[...] = en
    enb_ref[...] = en.astype(jnp.bfloat16)
    c_ref[...] = jnp.broadcast_to(jnp.sum(en * en, axis=1, keepdims=True),
                                  c_ref.shape)


def _vq_body(x_ref, enb_ref, c_ref, idx_ref, loss_ref,
             xn_scr, sx_scr, m_scr, ix_scr, acc_ref):
    ti = pl.program_id(0)
    cj = pl.program_id(1)
    nt = pl.num_programs(0)
    ncc = pl.num_programs(1)

    @pl.when(cj == 0)
    def _tile_init():
        x = x_ref[...]
        nrm = jnp.sqrt(jnp.sum(x * x, axis=1, keepdims=True))
        xn = x / jnp.maximum(nrm, EPS)
        xn_scr[...] = xn.astype(jnp.bfloat16)
        sx_scr[...] = jnp.sum(xn * xn, axis=1, keepdims=True)
        m_scr[...] = jnp.full(m_scr.shape, jnp.inf, jnp.float32)
        ix_scr[...] = jnp.zeros(ix_scr.shape, jnp.int32)

    @pl.when(jnp.logical_and(ti == 0, cj == 0))
    def _init():
        acc_ref[0, 0] = 0.0

    dot = lax.dot_general(
        xn_scr[...], enb_ref[...],
        (((1,), (1,)), ((), ())), preferred_element_type=jnp.float32)
    scores = (sx_scr[...] + c_ref[...]) - 2.0 * dot
    lm = jnp.min(scores, axis=1, keepdims=True)
    iot = lax.broadcasted_iota(jnp.int32, scores.shape, 1)
    li = jnp.min(jnp.where(scores == lm, iot, CC), axis=1,
                 keepdims=True) + cj * CC
    upd = lm < m_scr[...]
    ix_scr[...] = jnp.where(upd, li, ix_scr[...])
    m_scr[...] = jnp.where(upd, lm, m_scr[...])

    @pl.when(cj == ncc - 1)
    def _tile_fin():
        idx_ref[0, 0, :] = ix_scr[...].reshape(TN)
        acc_ref[0, 0] += jnp.sum(m_scr[...])

    @pl.when(jnp.logical_and(ti == nt - 1, cj == ncc - 1))
    def _fin():
        loss_ref[0, 0] = acc_ref[0, 0] * (1.0 + COMMIT) / (N * D)


def _gather_body(en_hbm, idx_hbm, out_hbm, idx_v, rows_v, sem):
    wid = lax.axis_index("s") * NC + lax.axis_index("c")
    pltpu.sync_copy(idx_hbm.at[pl.ds(wid * CH_PER_W, CH_PER_W)], idx_v)

    def body(j, carry):
        pltpu.async_copy(en_hbm.at[idx_v.at[j]], rows_v, sem).wait()
        pltpu.sync_copy(rows_v, out_hbm.at[pl.ds(wid * ROWS_PER_W + j * CK, CK)])
        return carry

    lax.fori_loop(0, CH_PER_W, body, 0)


def _gather_call(emb_n, idx2d):
    mesh = plsc.VectorSubcoreMesh(core_axis_name="c", subcore_axis_name="s")
    f = pl.kernel(
        _gather_body,
        out_type=jax.ShapeDtypeStruct((N, D), jnp.float32),
        mesh=mesh,
        scratch_types=[
            pltpu.VMEM((CH_PER_W, CK), jnp.int32),
            pltpu.VMEM((CK, D), jnp.float32),
            pltpu.SemaphoreType.DMA,
        ],
    )
    return f(emb_n, idx2d)


def kernel(x, embeddings):
    emb_n, emb_nb, c8 = pl.pallas_call(
        _norm_body,
        grid=(K // EB,),
        in_specs=[pl.BlockSpec((EB, D), lambda i: (i, 0))],
        out_specs=[
            pl.BlockSpec((EB, D), lambda i: (i, 0)),
            pl.BlockSpec((EB, D), lambda i: (i, 0)),
            pl.BlockSpec((EB, 128), lambda i: (i, 0)),
        ],
        out_shape=[
            jax.ShapeDtypeStruct((K, D), jnp.float32),
            jax.ShapeDtypeStruct((K, D), jnp.bfloat16),
            jax.ShapeDtypeStruct((K, 128), jnp.float32),
        ],
    )(embeddings)
    c_row = c8[:, :1].reshape(1, K)

    nt = N // TN
    idx3, loss = pl.pallas_call(
        _vq_body,
        grid=(nt, K // CC),
        in_specs=[
            pl.BlockSpec((TN, D), lambda i, j: (i, 0)),
            pl.BlockSpec((CC, D), lambda i, j: (j, 0)),
            pl.BlockSpec((1, CC), lambda i, j: (0, j)),
        ],
        out_specs=[
            pl.BlockSpec((1, 1, TN), lambda i, j: (i, 0, 0)),
            pl.BlockSpec((1, 1), lambda i, j: (0, 0), memory_space=pltpu.SMEM),
        ],
        out_shape=[
            jax.ShapeDtypeStruct((nt, 1, TN), jnp.int32),
            jax.ShapeDtypeStruct((1, 1), jnp.float32),
        ],
        scratch_shapes=[
            pltpu.VMEM((TN, D), jnp.bfloat16),
            pltpu.VMEM((TN, 1), jnp.float32),
            pltpu.VMEM((TN, 1), jnp.float32),
            pltpu.VMEM((TN, 1), jnp.int32),
            pltpu.SMEM((1, 1), jnp.float32),
        ],
    )(x, emb_nb, c_row)

    quantized = _gather_call(emb_n, idx3.reshape(N // CK, CK))
    return quantized, loss[0, 0], idx3.reshape(N)
